# baseline (speedup 1.0000x reference)
import functools

import jax
import jax.numpy as jnp
from jax import lax
from jax.experimental import pallas as pl
from jax.experimental.pallas import tpu as pltpu

L, B, E, H, F, NE = 2048, 1, 768, 12, 1024, 8
DH = E // H
T = L * B
NPAIR = 2 * T

BLK = 256
NPAD = NPAIR + NE * BLK
NB = NPAD // BLK
QBLK = 256


def _qkv_body(x_ref, w_ref, b_ref, o_ref):
    o_ref[0] = (
        jnp.dot(x_ref[...], w_ref[...].T, preferred_element_type=jnp.float32)
        + b_ref[0]
    )


def _qkv(x2d, in_proj_w, in_proj_b):
    return pl.pallas_call(
        _qkv_body,
        grid=(3 * H,),
        in_specs=[
            pl.BlockSpec((T, E), lambda i: (0, 0)),
            pl.BlockSpec((DH, E), lambda i: (i, 0)),
            pl.BlockSpec((1, 1, DH), lambda i: (i, 0, 0)),
        ],
        out_specs=pl.BlockSpec((1, T, DH), lambda i: (i, 0, 0)),
        out_shape=jax.ShapeDtypeStruct((3 * H, T, DH), jnp.float32),
    )(x2d, in_proj_w, in_proj_b.reshape(3 * H, 1, DH))


def _attn_body(q_ref, k_ref, v_ref, o_ref):
    q = q_ref[0]
    k = k_ref[0]
    v = v_ref[0]
    s = jax.lax.dot_general(
        q, k, (((1,), (1,)), ((), ())),
        preferred_element_type=jnp.float32) * (1.0 / (DH ** 0.5))
    m = jnp.max(s, axis=1, keepdims=True)
    e = jnp.exp(s - m)
    p = e / jnp.sum(e, axis=1, keepdims=True)
    o_ref[0] = jnp.dot(p, v, preferred_element_type=jnp.float32)


def _attention(qkv):
    nq = T // QBLK
    return pl.pallas_call(
        _attn_body,
        grid=(H, nq),
        in_specs=[
            pl.BlockSpec((1, QBLK, DH), lambda h, i: (h, i, 0)),
            pl.BlockSpec((1, T, DH), lambda h, i: (H + h, 0, 0)),
            pl.BlockSpec((1, T, DH), lambda h, i: (2 * H + h, 0, 0)),
        ],
        out_specs=pl.BlockSpec((1, QBLK, DH), lambda h, i: (h, i, 0)),
        out_shape=jax.ShapeDtypeStruct((H, T, DH), jnp.float32),
    )(qkv, qkv, qkv)


def _layer_norm_in(v, g, b):
    m = jnp.mean(v, axis=-1, keepdims=True)
    var = jnp.mean((v - m) ** 2, axis=-1, keepdims=True)
    return (v - m) * lax.rsqrt(var + 1e-5) * g + b


def _cumsum0(a):
    n = a.shape[0]
    sh = 1
    while sh < n:
        z = jnp.zeros((sh, a.shape[1]), a.dtype)
        a = a + jnp.concatenate([z, a[:-sh]], axis=0)
        sh *= 2
    return a


def _prefix_lanes(a):
    n = a.shape[1]
    sh = 1
    while sh < n:
        z = jnp.zeros((a.shape[0], sh), a.dtype)
        a = a + jnp.concatenate([z, a[:, :-sh]], axis=1)
        sh *= 2
    return a


def _router_body(o_ref, w_ref, b_ref, x_ref, g_ref, be_ref, gw_ref,
                 x1_ref, pos1_ref, pos2_ref, g1_ref, g2_ref, bexp_ref):
    proj = b_ref[...]
    for h in range(H):
        proj = proj + lax.dot_general(
            o_ref[h], w_ref[:, h * DH:(h + 1) * DH],
            (((1,), (1,)), ((), ())), preferred_element_type=jnp.float32)
    x1 = _layer_norm_in(x_ref[...] + proj, g_ref[...], be_ref[...])
    x1_ref[...] = x1

    logits = jnp.dot(x1, gw_ref[...], preferred_element_type=jnp.float32)
    lm = jnp.max(logits, axis=1, keepdims=True)
    el = jnp.exp(logits - lm)
    gates = el / jnp.sum(el, axis=1, keepdims=True)

    iota8 = lax.broadcasted_iota(jnp.int32, (T, NE), 1)
    g1v = jnp.max(gates, axis=1, keepdims=True)
    i1 = jnp.min(jnp.where(gates == g1v, iota8, NE), axis=1, keepdims=True)
    m1 = iota8 == i1
    gm = jnp.where(m1, -jnp.inf, gates)
    g2v = jnp.max(gm, axis=1, keepdims=True)
    i2 = jnp.min(jnp.where(gm == g2v, iota8, NE), axis=1, keepdims=True)
    m2 = iota8 == i2
    denom = g1v + g2v + 1e-9
    g1_ref[...] = g1v / denom
    g2_ref[...] = g2v / denom

    c1 = _cumsum0(m1.astype(jnp.int32))
    c2 = _cumsum0(m2.astype(jnp.int32))
    tot1 = c1[T - 1:T, :]
    total = tot1 + c2[T - 1:T, :]
    padded = ((total + (BLK - 1)) // BLK) * BLK
    poff = _prefix_lanes(padded) - padded

    m1i = m1.astype(jnp.int32)
    m2i = m2.astype(jnp.int32)
    pos1_ref[...] = jnp.sum(m1i * (poff + c1 - 1), axis=1, keepdims=True)
    pos2_ref[...] = jnp.sum(m2i * (poff + tot1 + c2 - 1), axis=1, keepdims=True)

    pboff = poff // BLK
    jiota = lax.broadcasted_iota(jnp.int32, (NB, NE), 0)
    bexp = jnp.sum((jiota >= pboff).astype(jnp.int32), axis=1, keepdims=True) - 1
    bexp_ref[...] = jnp.minimum(bexp, NE - 1)


def _router(o, out_proj_w, out_proj_b, x2d, ln1_g, ln1_b, gate_w):
    full = lambda r, c: pl.BlockSpec((r, c), lambda: (0, 0))
    return pl.pallas_call(
        _router_body,
        in_specs=[
            pl.BlockSpec((H, T, DH), lambda: (0, 0, 0)),
            full(E, E), full(1, E), full(T, E),
            full(1, E), full(1, E), full(E, NE),
        ],
        out_specs=[
            full(T, E), full(T, 1), full(T, 1), full(T, 1), full(T, 1),
            full(NB, 1),
        ],
        out_shape=[
            jax.ShapeDtypeStruct((T, E), jnp.float32),
            jax.ShapeDtypeStruct((T, 1), jnp.int32),
            jax.ShapeDtypeStruct((T, 1), jnp.int32),
            jax.ShapeDtypeStruct((T, 1), jnp.float32),
            jax.ShapeDtypeStruct((T, 1), jnp.float32),
            jax.ShapeDtypeStruct((NB, 1), jnp.int32),
        ],
    )(o, out_proj_w, out_proj_b.reshape(1, E), x2d,
      ln1_g.reshape(1, E), ln1_b.reshape(1, E), gate_w)


def _ffn_body(bexp_ref, xs_ref, sg_ref, w1_ref, w2_ref, o_ref):
    del bexp_ref
    h = jnp.dot(xs_ref[...], w1_ref[0], preferred_element_type=jnp.float32)
    h = jax.nn.gelu(h)
    o = jnp.dot(h, w2_ref[0], preferred_element_type=jnp.float32)
    o_ref[...] = o * sg_ref[...]


def _ffn(xs, sg, w1, w2, bexp):
    grid_spec = pltpu.PrefetchScalarGridSpec(
        num_scalar_prefetch=1,
        grid=(NB,),
        in_specs=[
            pl.BlockSpec((BLK, E), lambda j, be: (j, 0)),
            pl.BlockSpec((BLK, 1), lambda j, be: (j, 0)),
            pl.BlockSpec((1, E, F), lambda j, be: (be[j], 0, 0)),
            pl.BlockSpec((1, F, E), lambda j, be: (be[j], 0, 0)),
        ],
        out_specs=pl.BlockSpec((BLK, E), lambda j, be: (j, 0)),
    )
    return pl.pallas_call(
        _ffn_body,
        grid_spec=grid_spec,
        out_shape=jax.ShapeDtypeStruct((NPAD, E), jnp.float32),
    )(bexp, xs, sg, w1, w2)


def _ln2_body(x1_ref, ya_ref, yb_ref, g_ref, b_ref, o_ref):
    v = x1_ref[...] + ya_ref[...] + yb_ref[...]
    o_ref[...] = _layer_norm_in(v, g_ref[...], b_ref[...])


def _ln2(x1, ya, yb, ln2_g, ln2_b):
    full = lambda r, c: pl.BlockSpec((r, c), lambda: (0, 0))
    return pl.pallas_call(
        _ln2_body,
        in_specs=[full(T, E), full(T, E), full(T, E), full(1, E), full(1, E)],
        out_specs=full(T, E),
        out_shape=jax.ShapeDtypeStruct((T, E), jnp.float32),
    )(x1, ya, yb, ln2_g.reshape(1, E), ln2_b.reshape(1, E))


@jax.jit
def kernel(x, time, in_proj_w, in_proj_b, out_proj_w, out_proj_b,
           ln1_g, ln1_b, ln2_g, ln2_b, gate_w, w1, w2):
    del time
    x2d = x.reshape(T, E)

    qkv = _qkv(x2d, in_proj_w, in_proj_b)
    o = _attention(qkv)
    x1, pos1, pos2, g1, g2, bexp = _router(
        o, out_proj_w, out_proj_b, x2d, ln1_g, ln1_b, gate_w)

    p1 = pos1[:, 0]
    p2 = pos2[:, 0]
    tid = jnp.arange(T, dtype=jnp.int32)
    sorted_tid = (jnp.zeros((NPAD,), jnp.int32).at[p1].set(tid).at[p2].set(tid))
    sorted_g = (jnp.zeros((NPAD,), jnp.float32)
                .at[p1].set(g1[:, 0]).at[p2].set(g2[:, 0]))
    xs = jnp.take(x1, sorted_tid, axis=0)

    out_rows = _ffn(xs, sorted_g.reshape(NPAD, 1), w1, w2, bexp[:, 0])

    ya = jnp.take(out_rows, p1, axis=0)
    yb = jnp.take(out_rows, p2, axis=0)
    x2 = _ln2(x1, ya, yb, ln2_g, ln2_b)
    return x2.reshape(L, B, E)
